# baseline (speedup 1.0000x reference)
import functools

import jax
import jax.numpy as jnp
from jax import lax
from jax.experimental import pallas as pl
from jax.experimental.pallas import tpu as pltpu
from jax.experimental.pallas import tpu_sc as plsc

N = 38400
E = 614400
ICH = 128
HID = 64
HALF = 32
NPG = 128
NG = N // NPG
N2 = 2 * N

EC = 128
ER = E // EC
ERP = 5120
EPAD = ERP * EC - E
KB = 32

CR_TILE = ERP // 16
NB_G = CR_TILE // KB
ROWS_TILE = N // 16
ZR = 240

CR_W = ERP // 32
NB_D = CR_W // KB
DEG_TILE = N // 16

BM = 1536
GPB = BM // NPG
GRID = N // BM

_MESH = plsc.VectorSubcoreMesh(core_axis_name="c", subcore_axis_name="s")
_SC_PARAMS = pltpu.CompilerParams(use_tc_tiling_on_sc=False,
                                  needs_layout_passes=False)


def _deg_body(row_hbm, col_hbm, ew_hbm, out0_hbm, out1_hbm, rbuf, cbuf,
              ewbuf, zbuf, degacc, sem):
    c = lax.axis_index("c")
    s = lax.axis_index("s")

    @pl.loop(0, DEG_TILE // 16)
    def _zero(i):
        off = pl.multiple_of(i * 16, 16)
        zbuf[pl.ds(off, 16)] = jnp.zeros((16,), jnp.float32)

    pltpu.sync_copy(zbuf, degacc.at[pl.ds(s * DEG_TILE, DEG_TILE)])
    plsc.subcore_barrier()

    w = c * 16 + s
    r0w = w * CR_W

    @pl.loop(0, NB_D)
    def _blk(b):
        r0 = r0w + b * KB
        pltpu.sync_copy(row_hbm.at[pl.ds(r0, KB)], rbuf)
        pltpu.sync_copy(col_hbm.at[pl.ds(r0, KB)], cbuf)
        pltpu.sync_copy(ew_hbm.at[pl.ds(r0, KB)], ewbuf)

        @pl.loop(0, KB)
        def _row(j):
            for g in range(EC // 16):
                sl = pl.ds(g * 16, 16)
                r16 = rbuf[j, sl]
                c16 = cbuf[j, sl]
                w16 = ewbuf[j, sl]
                ewbuf[j, sl] = jnp.where(r16 != c16, w16, 0.0)
            pltpu.sync_copy(ewbuf.at[j], degacc.at[rbuf.at[j]], add=True)

    plsc.subcore_barrier()
    sl = pl.ds(s * DEG_TILE, DEG_TILE)
    pltpu.sync_copy(degacc.at[sl], zbuf)

    @pl.when(c == 0)
    def _w0():
        pltpu.sync_copy(zbuf, out0_hbm.at[sl])

    @pl.when(c == 1)
    def _w1():
        pltpu.sync_copy(zbuf, out1_hbm.at[sl])


_deg_call = pl.kernel(
    _deg_body,
    out_type=[jax.ShapeDtypeStruct((N,), jnp.float32),
              jax.ShapeDtypeStruct((N,), jnp.float32)],
    mesh=_MESH,
    scratch_types=[
        pltpu.VMEM((KB, EC), jnp.int32),
        pltpu.VMEM((KB, EC), jnp.int32),
        pltpu.VMEM((KB, EC), jnp.float32),
        pltpu.VMEM((DEG_TILE,), jnp.float32),
        pltpu.VMEM_SHARED((N,), jnp.float32),
        pltpu.SemaphoreType.DMA,
    ],
    compiler_params=_SC_PARAMS,
)


def _g_body(tab_hbm, row_hbm, col_hbm, ew_hbm, out_hbm, rbuf, cbuf, ewbuf,
            gibuf, rows, zbuf, acc, sem_g, sem_s):
    c = lax.axis_index("c")
    s = lax.axis_index("s")

    @pl.loop(0, ZR)
    def _zero(r):
        z16 = jnp.zeros((16,), jnp.float32)
        zbuf[r, pl.ds(0, 16)] = z16
        zbuf[r, pl.ds(16, 16)] = z16

    base = s * ROWS_TILE
    for t in range(ROWS_TILE // ZR):
        pltpu.sync_copy(zbuf, acc.at[pl.ds(base + t * ZR, ZR)])
    plsc.subcore_barrier()

    tile_r0 = s * CR_TILE

    @pl.loop(0, NB_G)
    def _blk(b):
        r0 = tile_r0 + b * KB
        pltpu.sync_copy(row_hbm.at[pl.ds(r0, KB)], rbuf)
        pltpu.sync_copy(col_hbm.at[pl.ds(r0, KB)], cbuf)
        pltpu.sync_copy(ew_hbm.at[pl.ds(r0, KB)], ewbuf)

        @pl.loop(0, KB)
        def _idx(j):
            for g in range(EC // 16):
                sl = pl.ds(g * 16, 16)
                r16 = rbuf[j, sl]
                c16 = cbuf[j, sl]
                w16 = ewbuf[j, sl]
                ewbuf[j, sl] = jnp.where(r16 != c16, w16, 0.0)
                gibuf[j, sl] = r16 * 2 + c

        pltpu.async_copy(tab_hbm.at[gibuf.at[0]], rows.at[0], sem_g)

        @pl.loop(0, KB)
        def _edge(j):
            slot = lax.rem(j, 2)
            pltpu.make_async_copy(tab_hbm.at[gibuf.at[j]], rows.at[slot],
                                  sem_g).wait()

            @pl.when(j >= 1)
            def _drain_prev_scatter():
                pltpu.make_async_copy(rows.at[1 - slot],
                                      acc.at[cbuf.at[j - 1]], sem_s).wait()

            @pl.when(j + 1 < KB)
            def _next():
                pltpu.async_copy(tab_hbm.at[gibuf.at[j + 1]],
                                 rows.at[1 - slot], sem_g)

            lo = pl.ds(0, 16)
            hi = pl.ds(16, 16)
            jsplat = jnp.full((16,), j, jnp.int32)
            for e in range(EC):
                wv = plsc.load_gather(ewbuf, [jsplat,
                                              jnp.full((16,), e, jnp.int32)])
                rows[slot, e, lo] = rows[slot, e, lo] * wv
                rows[slot, e, hi] = rows[slot, e, hi] * wv

            pltpu.async_copy(rows.at[slot], acc.at[cbuf.at[j]], sem_s,
                             add=True)

        pltpu.make_async_copy(rows.at[(KB - 1) % 2],
                              acc.at[cbuf.at[KB - 1]], sem_s).wait()

    plsc.subcore_barrier()
    for t in range(ROWS_TILE // ZR):
        sl = pl.ds(base + t * ZR, ZR)
        pltpu.sync_copy(acc.at[sl], zbuf)
        pltpu.sync_copy(zbuf, out_hbm.at[c, sl])


_g_call = pl.kernel(
    _g_body,
    out_type=jax.ShapeDtypeStruct((2, N, HALF), jnp.float32),
    mesh=_MESH,
    scratch_types=[
        pltpu.VMEM((KB, EC), jnp.int32),
        pltpu.VMEM((KB, EC), jnp.int32),
        pltpu.VMEM((KB, EC), jnp.float32),
        pltpu.VMEM((KB, EC), jnp.int32),
        pltpu.VMEM((2, EC, HALF), jnp.float32),
        pltpu.VMEM((ZR, HALF), jnp.float32),
        pltpu.VMEM_SHARED((N, HALF), jnp.float32),
        pltpu.SemaphoreType.DMA,
        pltpu.SemaphoreType.DMA,
    ],
    compiler_params=_SC_PARAMS,
)


def _prep_body(x_ref, d0_ref, d1_ref, w0_ref, w1_ref, w2_ref,
               v2s_ref, u1s_ref, z_ref, dis_ref):
    d = d0_ref[...] + d1_ref[...]
    dis = jnp.where(d > 0.0,
                    lax.rsqrt(jnp.where(d > 0.0, d, 1.0)), 0.0)
    xb = x_ref[...]
    m1 = jnp.dot(xb, w1_ref[...], preferred_element_type=jnp.float32)
    m2 = jnp.dot(xb, w2_ref[...], preferred_element_type=jnp.float32)
    z = jnp.dot(xb, w0_ref[...] - w2_ref[...],
                preferred_element_type=jnp.float32)
    v2s_ref[...] = dis * m2
    u1s_ref[...] = dis * m1
    z_ref[...] = z
    dis_ref[...] = dis


def _prep_call(x, d0, d1, W0, W1, W2):
    return pl.pallas_call(
        _prep_body,
        grid=(GRID,),
        in_specs=[
            pl.BlockSpec((BM, ICH), lambda i: (i, 0)),
            pl.BlockSpec((BM, 1), lambda i: (i, 0)),
            pl.BlockSpec((BM, 1), lambda i: (i, 0)),
            pl.BlockSpec((ICH, HID), lambda i: (0, 0)),
            pl.BlockSpec((ICH, HID), lambda i: (0, 0)),
            pl.BlockSpec((ICH, HID), lambda i: (0, 0)),
        ],
        out_specs=[
            pl.BlockSpec((BM, HID), lambda i: (i, 0)),
            pl.BlockSpec((BM, HID), lambda i: (i, 0)),
            pl.BlockSpec((BM, HID), lambda i: (i, 0)),
            pl.BlockSpec((BM, 1), lambda i: (i, 0)),
        ],
        out_shape=[
            jax.ShapeDtypeStruct((N, HID), jnp.float32),
            jax.ShapeDtypeStruct((N, HID), jnp.float32),
            jax.ShapeDtypeStruct((N, HID), jnp.float32),
            jax.ShapeDtypeStruct((N, 1), jnp.float32),
        ],
    )(x, d0, d1, W0, W1, W2)


def _mid_body(u1s_ref, g2lo_ref, g2hi_ref, dis_ref, p1_ref):
    dis = dis_ref[...]
    d2 = 2.0 * dis * dis
    g2 = jnp.concatenate([g2lo_ref[0], g2hi_ref[0]], axis=1)
    p1_ref[...] = u1s_ref[...] - d2 * g2


def _mid_call(u1s, gs2, dis):
    return pl.pallas_call(
        _mid_body,
        grid=(GRID,),
        in_specs=[
            pl.BlockSpec((BM, HID), lambda i: (i, 0)),
            pl.BlockSpec((1, BM, HALF), lambda i: (0, i, 0)),
            pl.BlockSpec((1, BM, HALF), lambda i: (1, i, 0)),
            pl.BlockSpec((BM, 1), lambda i: (i, 0)),
        ],
        out_specs=pl.BlockSpec((BM, HID), lambda i: (i, 0)),
        out_shape=jax.ShapeDtypeStruct((N, HID), jnp.float32),
    )(u1s, gs2, gs2, dis)


def _out_body(z_ref, g1lo_ref, g1hi_ref, dis_ref, bc_ref, wl_ref, m_ref,
              ones_ref, bl_ref, y_ref):
    g1 = jnp.concatenate([g1lo_ref[0], g1hi_ref[0]], axis=1)
    out = z_ref[...] - dis_ref[...] * g1 + bc_ref[...]
    h = jnp.maximum(out, 0.0)
    q = h * wl_ref[...]
    t = jnp.dot(m_ref[...], q, preferred_element_type=jnp.float32)
    y = jnp.dot(t, ones_ref[...], preferred_element_type=jnp.float32)
    y_ref[0] = jax.nn.sigmoid(y + bl_ref[...])


def _out_call(z, gs1, dis, bc, wl_tiled, m_mask, ones64, bl):
    return pl.pallas_call(
        _out_body,
        grid=(GRID,),
        in_specs=[
            pl.BlockSpec((BM, HID), lambda i: (i, 0)),
            pl.BlockSpec((1, BM, HALF), lambda i: (0, i, 0)),
            pl.BlockSpec((1, BM, HALF), lambda i: (1, i, 0)),
            pl.BlockSpec((BM, 1), lambda i: (i, 0)),
            pl.BlockSpec((1, HID), lambda i: (0, 0)),
            pl.BlockSpec((BM, HID), lambda i: (0, 0)),
            pl.BlockSpec((GPB, BM), lambda i: (0, 0)),
            pl.BlockSpec((HID, 1), lambda i: (0, 0)),
            pl.BlockSpec((1, 1), lambda i: (0, 0)),
        ],
        out_specs=pl.BlockSpec((1, GPB, 1), lambda i: (i, 0, 0)),
        out_shape=jax.ShapeDtypeStruct((GRID, GPB, 1), jnp.float32),
    )(z, gs1, gs1, dis, bc, wl_tiled, m_mask, ones64, bl)


def kernel(x, edge_index, edge_weight, batch, W0, W1, W2, b_conv, W_lin,
           b_lin):
    ipad = jnp.zeros((EPAD,), jnp.int32)
    row2 = jnp.concatenate([edge_index[0], ipad]).reshape(ERP, EC)
    col2 = jnp.concatenate([edge_index[1], ipad]).reshape(ERP, EC)
    ew2 = jnp.concatenate([edge_weight.astype(jnp.float32),
                           jnp.zeros((EPAD,), jnp.float32)]).reshape(ERP, EC)

    deg0, deg1 = _deg_call(row2, col2, ew2)
    d0 = deg0.reshape(N, 1)
    d1 = deg1.reshape(N, 1)

    v2s, u1s, z, dis = _prep_call(x, d0, d1, W0, W1, W2)

    gs2 = _g_call(v2s.reshape(N2, HALF), row2, col2, ew2)
    p1 = _mid_call(u1s, gs2, dis)
    gs1 = _g_call(p1.reshape(N2, HALF), row2, col2, ew2)

    bc = b_conv.astype(jnp.float32).reshape(1, HID)
    wl = W_lin.astype(jnp.float32).reshape(NPG, HID)
    wl_tiled = jnp.tile(wl, (GPB, 1))
    m_mask = jnp.kron(jnp.eye(GPB, dtype=jnp.float32),
                      jnp.ones((1, NPG), jnp.float32))
    ones64 = jnp.ones((HID, 1), jnp.float32)
    bl = b_lin.astype(jnp.float32).reshape(1, 1)

    y = _out_call(z, gs1, dis, bc, wl_tiled, m_mask, ones64, bl)
    return y.reshape(NG, 1)


# speedup vs baseline: 1.6639x; 1.6639x over previous
import functools

import jax
import jax.numpy as jnp
from jax import lax
from jax.experimental import pallas as pl
from jax.experimental.pallas import tpu as pltpu
from jax.experimental.pallas import tpu_sc as plsc

N = 38400
E = 614400
ICH = 128
HID = 64
HALF = 32
NPG = 128
NG = N // NPG
N2 = 2 * N

EC = 256
ER = E // EC
ERP = 2560
EPAD = ERP * EC - E
KB = 16

CR_TILE = ERP // 16
NB_G = CR_TILE // KB
ROWS_TILE = N // 16
ZR = 120

CR_W = ERP // 32
NB_D = CR_W // KB
DEG_TILE = N // 16

BM = 1536
GPB = BM // NPG
GRID = N // BM

_MESH = plsc.VectorSubcoreMesh(core_axis_name="c", subcore_axis_name="s")
_SC_PARAMS = pltpu.CompilerParams(use_tc_tiling_on_sc=False,
                                  needs_layout_passes=False)


def _deg_body(row_hbm, col_hbm, ew_hbm, out0_hbm, out1_hbm, rbuf, cbuf,
              ewbuf, zbuf, degacc, sem):
    c = lax.axis_index("c")
    s = lax.axis_index("s")

    @pl.loop(0, DEG_TILE // 16)
    def _zero(i):
        off = pl.multiple_of(i * 16, 16)
        zbuf[pl.ds(off, 16)] = jnp.zeros((16,), jnp.float32)

    pltpu.sync_copy(zbuf, degacc.at[pl.ds(s * DEG_TILE, DEG_TILE)])
    plsc.subcore_barrier()

    w = c * 16 + s
    r0w = w * CR_W

    @pl.loop(0, NB_D)
    def _blk(b):
        r0 = r0w + b * KB
        pltpu.sync_copy(row_hbm.at[pl.ds(r0, KB)], rbuf)
        pltpu.sync_copy(col_hbm.at[pl.ds(r0, KB)], cbuf)
        pltpu.sync_copy(ew_hbm.at[pl.ds(r0, KB)], ewbuf)

        @pl.loop(0, KB)
        def _row(j):
            for g in range(EC // 16):
                sl = pl.ds(g * 16, 16)
                r16 = rbuf[j, sl]
                c16 = cbuf[j, sl]
                w16 = ewbuf[j, sl]
                ewbuf[j, sl] = jnp.where(r16 != c16, w16, 0.0)
            pltpu.sync_copy(ewbuf.at[j], degacc.at[rbuf.at[j]], add=True)

    plsc.subcore_barrier()
    sl = pl.ds(s * DEG_TILE, DEG_TILE)
    pltpu.sync_copy(degacc.at[sl], zbuf)

    @pl.when(c == 0)
    def _w0():
        pltpu.sync_copy(zbuf, out0_hbm.at[sl])

    @pl.when(c == 1)
    def _w1():
        pltpu.sync_copy(zbuf, out1_hbm.at[sl])


_deg_call = pl.kernel(
    _deg_body,
    out_type=[jax.ShapeDtypeStruct((N,), jnp.float32),
              jax.ShapeDtypeStruct((N,), jnp.float32)],
    mesh=_MESH,
    scratch_types=[
        pltpu.VMEM((KB, EC), jnp.int32),
        pltpu.VMEM((KB, EC), jnp.int32),
        pltpu.VMEM((KB, EC), jnp.float32),
        pltpu.VMEM((DEG_TILE,), jnp.float32),
        pltpu.VMEM_SHARED((N,), jnp.float32),
        pltpu.SemaphoreType.DMA,
    ],
    compiler_params=_SC_PARAMS,
)


def _g_body(tab_hbm, row_hbm, col_hbm, ew_hbm, out_hbm, rbuf, cbuf, ewbuf,
            gibuf, rows, msg, zbuf, acc, sg0, sg1, sg2, sg3, ss0, ss1, ss2,
            ss3):
    sem_g = [sg0, sg1, sg2, sg3]
    sem_s = [ss0, ss1, ss2, ss3]
    c = lax.axis_index("c")
    s = lax.axis_index("s")

    @pl.loop(0, ZR)
    def _zero(r):
        z16 = jnp.zeros((16,), jnp.float32)
        zbuf[r, pl.ds(0, 16)] = z16
        zbuf[r, pl.ds(16, 16)] = z16

    base = s * ROWS_TILE
    for t in range(ROWS_TILE // ZR):
        pltpu.sync_copy(zbuf, acc.at[pl.ds(base + t * ZR, ZR)])
    plsc.subcore_barrier()

    tile_r0 = s * CR_TILE

    @pl.loop(0, NB_G)
    def _blk(b):
        r0 = tile_r0 + b * KB
        pltpu.sync_copy(row_hbm.at[pl.ds(r0, KB)], rbuf)
        pltpu.sync_copy(col_hbm.at[pl.ds(r0, KB)], cbuf)
        pltpu.sync_copy(ew_hbm.at[pl.ds(r0, KB)], ewbuf)

        @pl.loop(0, KB)
        def _idx(j):
            for g in range(EC // 16):
                sl = pl.ds(g * 16, 16)
                r16 = rbuf[j, sl]
                c16 = cbuf[j, sl]
                w16 = ewbuf[j, sl]
                ewbuf[j, sl] = jnp.where(r16 != c16, w16, 0.0)
                gibuf[j, sl] = r16 * 2 + c

        for p in range(3):
            pltpu.async_copy(tab_hbm.at[gibuf.at[p]], rows.at[p], sem_g[p])

        @pl.loop(0, KB, step=4)
        def _edge(j):
            lo = pl.ds(0, 16)
            hi = pl.ds(16, 16)
            for ss in range(4):
                jj = j + ss
                pltpu.make_async_copy(tab_hbm.at[gibuf.at[jj]], rows.at[ss],
                                      sem_g[ss]).wait()

                @pl.when(jj >= 2)
                def _drain_scatter():
                    pltpu.make_async_copy(msg.at[ss % 2],
                                          acc.at[cbuf.at[jj - 2]],
                                          sem_s[ss % 2]).wait()

                nslot = (ss + 3) % 4

                @pl.when(jj + 3 < KB)
                def _next():
                    pltpu.async_copy(tab_hbm.at[gibuf.at[jj + 3]],
                                     rows.at[nslot], sem_g[nslot])

                jsplat = jnp.full((16,), jj, jnp.int32)

                @plsc.parallel_loop(0, EC, unroll=8)
                def _scale(e):
                    esp = jnp.full((16,), e, jnp.int32)
                    wv = plsc.load_gather(ewbuf, [jsplat, esp])
                    packed = rows[ss, e, :]
                    a, b = plsc.unpack(packed,
                                       format=plsc.PackFormat.INTERLEAVED,
                                       preferred_element_type=jnp.float32)
                    msg[ss % 2, e, lo] = a * wv
                    msg[ss % 2, e, hi] = b * wv

                pltpu.async_copy(msg.at[ss % 2], acc.at[cbuf.at[jj]],
                                 sem_s[ss % 2], add=True)

        for p in range(2):
            pltpu.make_async_copy(msg.at[p], acc.at[cbuf.at[KB - 2 + p]],
                                  sem_s[p]).wait()

    plsc.subcore_barrier()
    for t in range(ROWS_TILE // ZR):
        sl = pl.ds(base + t * ZR, ZR)
        pltpu.sync_copy(acc.at[sl], zbuf)
        pltpu.sync_copy(zbuf, out_hbm.at[c, sl])


_g_call = pl.kernel(
    _g_body,
    out_type=jax.ShapeDtypeStruct((2, N, HALF), jnp.float32),
    mesh=_MESH,
    scratch_types=[
        pltpu.VMEM((KB, EC), jnp.int32),
        pltpu.VMEM((KB, EC), jnp.int32),
        pltpu.VMEM((KB, EC), jnp.float32),
        pltpu.VMEM((KB, EC), jnp.int32),
        pltpu.VMEM((4, EC, HALF), jnp.bfloat16),
        pltpu.VMEM((2, EC, HALF), jnp.float32),
        pltpu.VMEM((ZR, HALF), jnp.float32),
        pltpu.VMEM_SHARED((N, HALF), jnp.float32),
    ] + [pltpu.SemaphoreType.DMA] * 8,
    compiler_params=_SC_PARAMS,
)


def _prep_body(x_ref, d0_ref, d1_ref, w0_ref, w1_ref, w2_ref, p_ref,
               tb_ref, u1s_ref, z_ref, dis_ref):
    d = d0_ref[...] + d1_ref[...]
    dis = jnp.where(d > 0.0,
                    lax.rsqrt(jnp.where(d > 0.0, d, 1.0)), 0.0)
    xb = x_ref[...]
    m1 = jnp.dot(xb, w1_ref[...], preferred_element_type=jnp.float32)
    m2 = jnp.dot(xb, w2_ref[...], preferred_element_type=jnp.float32)
    z = jnp.dot(xb, w0_ref[...] - w2_ref[...],
                preferred_element_type=jnp.float32)
    m2p = jnp.dot(m2, p_ref[...], preferred_element_type=jnp.float32)
    tb_ref[...] = (dis * m2p).astype(jnp.bfloat16)
    u1s_ref[...] = dis * m1
    z_ref[...] = z
    dis_ref[...] = dis


def _prep_call(x, d0, d1, W0, W1, W2, P):
    return pl.pallas_call(
        _prep_body,
        grid=(GRID,),
        in_specs=[
            pl.BlockSpec((BM, ICH), lambda i: (i, 0)),
            pl.BlockSpec((BM, 1), lambda i: (i, 0)),
            pl.BlockSpec((BM, 1), lambda i: (i, 0)),
            pl.BlockSpec((ICH, HID), lambda i: (0, 0)),
            pl.BlockSpec((ICH, HID), lambda i: (0, 0)),
            pl.BlockSpec((ICH, HID), lambda i: (0, 0)),
            pl.BlockSpec((HID, HID), lambda i: (0, 0)),
        ],
        out_specs=[
            pl.BlockSpec((BM, HID), lambda i: (i, 0)),
            pl.BlockSpec((BM, HID), lambda i: (i, 0)),
            pl.BlockSpec((BM, HID), lambda i: (i, 0)),
            pl.BlockSpec((BM, 1), lambda i: (i, 0)),
        ],
        out_shape=[
            jax.ShapeDtypeStruct((N, HID), jnp.bfloat16),
            jax.ShapeDtypeStruct((N, HID), jnp.float32),
            jax.ShapeDtypeStruct((N, HID), jnp.float32),
            jax.ShapeDtypeStruct((N, 1), jnp.float32),
        ],
    )(x, d0, d1, W0, W1, W2, P)


def _mid_body(u1s_ref, g2lo_ref, g2hi_ref, dis_ref, p_ref, tb_ref):
    dis = dis_ref[...]
    d2 = 2.0 * dis * dis
    g2 = jnp.concatenate([g2lo_ref[0], g2hi_ref[0]], axis=1)
    p1 = u1s_ref[...] - d2 * g2
    p1p = jnp.dot(p1, p_ref[...], preferred_element_type=jnp.float32)
    tb_ref[...] = p1p.astype(jnp.bfloat16)


def _mid_call(u1s, gs2, dis, P):
    return pl.pallas_call(
        _mid_body,
        grid=(GRID,),
        in_specs=[
            pl.BlockSpec((BM, HID), lambda i: (i, 0)),
            pl.BlockSpec((1, BM, HALF), lambda i: (0, i, 0)),
            pl.BlockSpec((1, BM, HALF), lambda i: (1, i, 0)),
            pl.BlockSpec((BM, 1), lambda i: (i, 0)),
            pl.BlockSpec((HID, HID), lambda i: (0, 0)),
        ],
        out_specs=pl.BlockSpec((BM, HID), lambda i: (i, 0)),
        out_shape=jax.ShapeDtypeStruct((N, HID), jnp.bfloat16),
    )(u1s, gs2, gs2, dis, P)


def _out_body(z_ref, g1lo_ref, g1hi_ref, dis_ref, bc_ref, wl_ref, m_ref,
              ones_ref, bl_ref, y_ref):
    g1 = jnp.concatenate([g1lo_ref[0], g1hi_ref[0]], axis=1)
    out = z_ref[...] - dis_ref[...] * g1 + bc_ref[...]
    h = jnp.maximum(out, 0.0)
    q = h * wl_ref[...]
    t = jnp.dot(m_ref[...], q, preferred_element_type=jnp.float32)
    y = jnp.dot(t, ones_ref[...], preferred_element_type=jnp.float32)
    y_ref[0] = jax.nn.sigmoid(y + bl_ref[...])


def _out_call(z, gs1, dis, bc, wl_tiled, m_mask, ones64, bl):
    return pl.pallas_call(
        _out_body,
        grid=(GRID,),
        in_specs=[
            pl.BlockSpec((BM, HID), lambda i: (i, 0)),
            pl.BlockSpec((1, BM, HALF), lambda i: (0, i, 0)),
            pl.BlockSpec((1, BM, HALF), lambda i: (1, i, 0)),
            pl.BlockSpec((BM, 1), lambda i: (i, 0)),
            pl.BlockSpec((1, HID), lambda i: (0, 0)),
            pl.BlockSpec((BM, HID), lambda i: (0, 0)),
            pl.BlockSpec((GPB, BM), lambda i: (0, 0)),
            pl.BlockSpec((HID, 1), lambda i: (0, 0)),
            pl.BlockSpec((1, 1), lambda i: (0, 0)),
        ],
        out_specs=pl.BlockSpec((1, GPB, 1), lambda i: (i, 0, 0)),
        out_shape=jax.ShapeDtypeStruct((GRID, GPB, 1), jnp.float32),
    )(z, gs1, gs1, dis, bc, wl_tiled, m_mask, ones64, bl)


def kernel(x, edge_index, edge_weight, batch, W0, W1, W2, b_conv, W_lin,
           b_lin):
    ipad = jnp.zeros((EPAD,), jnp.int32)
    row2 = jnp.concatenate([edge_index[0], ipad]).reshape(ERP, EC)
    col2 = jnp.concatenate([edge_index[1], ipad]).reshape(ERP, EC)
    ew2 = jnp.concatenate([edge_weight.astype(jnp.float32),
                           jnp.zeros((EPAD,), jnp.float32)]).reshape(ERP, EC)

    deg0, deg1 = _deg_call(row2, col2, ew2)
    d0 = deg0.reshape(N, 1)
    d1 = deg1.reshape(N, 1)

    src = [(p // 32) * 32 + (p % 32 % 2) * 16 + (p % 32) // 2
           for p in range(HID)]
    P = jax.nn.one_hot(jnp.array(src, jnp.int32), HID,
                       dtype=jnp.float32).T

    tb2, u1s, z, dis = _prep_call(x, d0, d1, W0, W1, W2, P)

    gs2 = _g_call(tb2.reshape(N2, HALF), row2, col2, ew2)
    tb1 = _mid_call(u1s, gs2, dis, P)
    gs1 = _g_call(tb1.reshape(N2, HALF), row2, col2, ew2)

    bc = b_conv.astype(jnp.float32).reshape(1, HID)
    wl = W_lin.astype(jnp.float32).reshape(NPG, HID)
    wl_tiled = jnp.tile(wl, (GPB, 1))
    m_mask = jnp.kron(jnp.eye(GPB, dtype=jnp.float32),
                      jnp.ones((1, NPG), jnp.float32))
    ones64 = jnp.ones((HID, 1), jnp.float32)
    bl = b_lin.astype(jnp.float32).reshape(1, 1)

    y = _out_call(z, gs1, dis, bc, wl_tiled, m_mask, ones64, bl)
    return y.reshape(NG, 1)
